# baseline (speedup 1.0000x reference)
import jax
import jax.numpy as jnp
from jax.experimental import pallas as pl
from jax.experimental.pallas import tpu as pltpu
from jax.experimental.pallas import tpu_sc as plsc

N = 100000
STATIC_DIM = 128
DYNAMIC_DIM = 128
OUT_DIM = STATIC_DIM + DYNAMIC_DIM

TC_ROWS = 60000
TC_BLOCK = 4000
SC_ROWS = N - TC_ROWS

SC_CHUNK = 200
N_SC_CHUNKS = SC_ROWS // SC_CHUNK
N_WORKERS = 32


def _tc_block_body(a_ref, b_ref, o_ref, sem_a, sem_b):
    ca = pltpu.make_async_copy(a_ref, o_ref.at[:, pl.ds(0, STATIC_DIM)], sem_a)
    cb = pltpu.make_async_copy(
        b_ref, o_ref.at[:, pl.ds(STATIC_DIM, DYNAMIC_DIM)], sem_b)
    ca.start()
    cb.start()
    ca.wait()
    cb.wait()


def _sc_body(a_hbm, b_hbm, o_hbm):
    def scoped(abuf, bbuf, sems):
        w = jax.lax.axis_index("core") * 16 + jax.lax.axis_index("subcore")

        @pl.loop(w, N_SC_CHUNKS, step=N_WORKERS)
        def _(c):
            src = pl.ds(pl.multiple_of(TC_ROWS + c * SC_CHUNK, 8), SC_CHUNK)
            dst = pl.ds(pl.multiple_of(c * SC_CHUNK, 8), SC_CHUNK)
            ia = pltpu.make_async_copy(a_hbm.at[src, :], abuf, sems.at[0])
            ib = pltpu.make_async_copy(b_hbm.at[src, :], bbuf, sems.at[1])
            ia.start()
            ib.start()
            ia.wait()
            ib.wait()
            oa = pltpu.make_async_copy(
                abuf, o_hbm.at[dst, pl.ds(0, STATIC_DIM)], sems.at[0])
            ob = pltpu.make_async_copy(
                bbuf, o_hbm.at[dst, pl.ds(STATIC_DIM, DYNAMIC_DIM)],
                sems.at[1])
            oa.start()
            ob.start()
            oa.wait()
            ob.wait()

    pl.run_scoped(
        scoped,
        pltpu.VMEM((SC_CHUNK, STATIC_DIM), jnp.float32),
        pltpu.VMEM((SC_CHUNK, DYNAMIC_DIM), jnp.float32),
        pltpu.SemaphoreType.DMA((2,)),
    )


def kernel(static_emb, dynamic_emb):
    tc_out = pl.pallas_call(
        _tc_block_body,
        grid=(TC_ROWS // TC_BLOCK,),
        in_specs=[
            pl.BlockSpec((TC_BLOCK, STATIC_DIM), lambda i: (i, 0)),
            pl.BlockSpec((TC_BLOCK, DYNAMIC_DIM), lambda i: (i, 0)),
        ],
        out_specs=pl.BlockSpec((TC_BLOCK, OUT_DIM), lambda i: (i, 0)),
        out_shape=jax.ShapeDtypeStruct((TC_ROWS, OUT_DIM), jnp.float32),
        scratch_shapes=[pltpu.SemaphoreType.DMA, pltpu.SemaphoreType.DMA],
    )(static_emb, dynamic_emb)

    sc_mesh = plsc.VectorSubcoreMesh(core_axis_name="core",
                                     subcore_axis_name="subcore")
    sc_out = pl.kernel(
        _sc_body,
        out_type=jax.ShapeDtypeStruct((SC_ROWS, OUT_DIM), jnp.float32),
        mesh=sc_mesh,
    )(static_emb, dynamic_emb)

    return jnp.concatenate([tc_out, sc_out], axis=0)


# speedup vs baseline: 1.8522x; 1.8522x over previous
import jax
import jax.numpy as jnp
from jax.experimental import pallas as pl
from jax.experimental.pallas import tpu as pltpu

N = 100000
STATIC_DIM = 128
DYNAMIC_DIM = 128
OUT_DIM = STATIC_DIM + DYNAMIC_DIM
CHUNK = 1000
N_CHUNKS = N // CHUNK
N_SLOTS = 4
N_ITERS = N_CHUNKS // N_SLOTS


def _rows(c):
    return pl.ds(pl.multiple_of(c * CHUNK, 8), CHUNK)


def _body(a_hbm, b_hbm, o_hbm, obuf, in_sems, out_sems):
    def in_copies(c, s):
        return (
            pltpu.make_async_copy(
                a_hbm.at[_rows(c), :],
                obuf.at[s, :, pl.ds(0, STATIC_DIM)], in_sems.at[s, 0]),
            pltpu.make_async_copy(
                b_hbm.at[_rows(c), :],
                obuf.at[s, :, pl.ds(STATIC_DIM, DYNAMIC_DIM)],
                in_sems.at[s, 1]),
        )

    def out_copy(c, s):
        return pltpu.make_async_copy(obuf.at[s], o_hbm.at[_rows(c), :],
                                      out_sems.at[s])

    @pl.loop(0, N_ITERS)
    def _(q):
        for s in range(N_SLOTS):
            c = q * N_SLOTS + s

            @pl.when(c >= N_SLOTS)
            def _():
                out_copy(c - N_SLOTS, s).wait()

            for cp in in_copies(c, s):
                cp.start()
        for s in range(N_SLOTS):
            c = q * N_SLOTS + s
            for cp in in_copies(c, s):
                cp.wait()
            out_copy(c, s).start()

    for s in range(N_SLOTS):
        out_copy(N_CHUNKS - N_SLOTS + s, s).wait()


def kernel(static_emb, dynamic_emb):
    return pl.pallas_call(
        _body,
        in_specs=[
            pl.BlockSpec(memory_space=pltpu.MemorySpace.HBM),
            pl.BlockSpec(memory_space=pltpu.MemorySpace.HBM),
        ],
        out_specs=pl.BlockSpec(memory_space=pltpu.MemorySpace.HBM),
        out_shape=jax.ShapeDtypeStruct((N, OUT_DIM), jnp.float32),
        scratch_shapes=[
            pltpu.VMEM((N_SLOTS, CHUNK, OUT_DIM), jnp.float32),
            pltpu.SemaphoreType.DMA((N_SLOTS, 2)),
            pltpu.SemaphoreType.DMA((N_SLOTS,)),
        ],
    )(static_emb, dynamic_emb)


# speedup vs baseline: 2.2913x; 1.2370x over previous
import jax
import jax.numpy as jnp
from jax.experimental import pallas as pl
from jax.experimental.pallas import tpu as pltpu

N = 100000
STATIC_DIM = 128
DYNAMIC_DIM = 128
OUT_DIM = STATIC_DIM + DYNAMIC_DIM
BLOCK_N = 10000


def _body(a_ref, b_ref, o_ref, sem_a, sem_b):
    ca = pltpu.make_async_copy(a_ref, o_ref.at[:, pl.ds(0, STATIC_DIM)], sem_a)
    cb = pltpu.make_async_copy(
        b_ref, o_ref.at[:, pl.ds(STATIC_DIM, DYNAMIC_DIM)], sem_b)
    ca.start()
    cb.start()
    ca.wait()
    cb.wait()


def kernel(static_emb, dynamic_emb):
    return pl.pallas_call(
        _body,
        grid=(N // BLOCK_N,),
        in_specs=[
            pl.BlockSpec((BLOCK_N, STATIC_DIM), lambda i: (i, 0)),
            pl.BlockSpec((BLOCK_N, DYNAMIC_DIM), lambda i: (i, 0)),
        ],
        out_specs=pl.BlockSpec((BLOCK_N, OUT_DIM), lambda i: (i, 0)),
        out_shape=jax.ShapeDtypeStruct((N, OUT_DIM), jnp.float32),
        scratch_shapes=[pltpu.SemaphoreType.DMA, pltpu.SemaphoreType.DMA],
    )(static_emb, dynamic_emb)
